# baseline (speedup 1.0000x reference)
import functools

import jax
import jax.numpy as jnp
from jax import lax
from jax.experimental import pallas as pl
from jax.experimental.pallas import tpu as pltpu
from jax.experimental.pallas import tpu_sc as plsc

OX, OY, OZ = 8, 8, 8
VOX = OX * OY * OZ
L = 16
NEG = -3.0e38


def _pool_sc(params16, pts_t, pts_feature, n_rois):
    P, C = pts_feature.shape
    assert C == 2 * L
    assert n_rois % 32 == 0
    rois_per_w = n_rois // 32
    assert rois_per_w == 2
    pbits = (P - 1).bit_length()
    mesh = plsc.VectorSubcoreMesh(core_axis_name="c", subcore_axis_name="s")

    @functools.partial(
        pl.kernel,
        out_type=jax.ShapeDtypeStruct((n_rois, VOX * C), jnp.float32),
        mesh=mesh,
        scratch_types=[
            pltpu.VMEM((2, 12), jnp.float32),
            pltpu.VMEM((3 * P,), jnp.float32),
            pltpu.VMEM((2, VOX * C), jnp.float32),
            pltpu.VMEM((2, VOX), jnp.int32),
            pltpu.VMEM((2, P), jnp.int32),
            pltpu.VMEM((L, C), jnp.float32),
            pltpu.VMEM((L,), jnp.int32),
            pltpu.SemaphoreType.DMA,
        ],
        compiler_params=pltpu.CompilerParams(
            needs_layout_passes=False, use_tc_tiling_on_sc=False),
    )
    def body(params_hbm, pts_hbm, feat_hbm, out_hbm,
             prm, ptsv, grid, hit, lst, rows, idxb, sem):
        wid = lax.axis_index("s") * 2 + lax.axis_index("c")
        r0 = wid * 2
        with jax.named_scope("stage"):
            pltpu.sync_copy(pts_hbm, ptsv)
            pltpu.sync_copy(params_hbm.at[pl.ds(r0, 2)], prm)

        lane = lax.iota(jnp.int32, L)
        lane3 = lane * 3
        zf = jnp.zeros((L,), jnp.float32)
        zi = jnp.zeros((L,), jnp.int32)

        def zero_grid(i, carry):
            base = i * (4 * L)
            for r in range(2):
                for u in range(4):
                    grid[r, pl.ds(base + u * L, L)] = zf
            return carry

        with jax.named_scope("zero"):
            lax.fori_loop(0, VOX * C // (4 * L), zero_grid, 0)

        def zero_hit(i, carry):
            hit[0, pl.ds(i * L, L)] = zi
            hit[1, pl.ds(i * L, L)] = zi
            return carry

        lax.fori_loop(0, VOX // L, zero_hit, 0)

        def gp(r, k):
            return plsc.load_gather(
                prm, [jnp.full((L,), r, jnp.int32),
                      jnp.full((L,), k, jnp.int32)])

        pr = [[gp(r, k) for k in range(12)] for r in range(2)]

        def clampi(t, hi):
            ti = t.astype(jnp.int32)
            return jnp.minimum(jnp.maximum(ti, 0), hi)

        def p1_body(i, cnts):
            new = []
            for sub in range(2):
                ii = i * 2 + sub
                ix = ii * (3 * L) + lane3
                x = plsc.load_gather(ptsv, [ix])
                y = plsc.load_gather(ptsv, [ix + 1])
                z = plsc.load_gather(ptsv, [ix + 2])
                pid = ii * L + lane

                pxs, pys, pzs, zoks = [], [], [], []
                for r in range(2):
                    cx, cy, cz, ca, sa, hdx, hdy, dz, xre, yre, zre, mr2 = pr[r]
                    px = x - cx
                    py = y - cy
                    pz = z - cz
                    zok = (pz > 0.0) & (pz < dz)
                    pxs.append(px); pys.append(py); pzs.append(pz)
                    zoks.append(zok)

                cs = cnts if sub == 0 else new
                outs = []
                for r in range(2):
                    cx, cy, cz, ca, sa, hdx, hdy, dz, xre, yre, zre, mr2 = pr[r]
                    px, py, pz = pxs[r], pys[r], pzs[r]
                    lx = px * ca - py * sa
                    ly = px * sa + py * ca
                    inb = ((jnp.abs(lx) < hdx) & (jnp.abs(ly) < hdy)
                           & zoks[r])
                    xi = clampi((lx + hdx) / xre, OX - 1)
                    yi = clampi((ly + hdy) / yre, OY - 1)
                    zvi = clampi(pz / zre, OZ - 1)
                    vox = (xi * OY + yi) * OZ + zvi
                    pk = (vox << pbits) | pid
                    mi = inb.astype(jnp.int32)
                    incl = plsc.cumsum(mi)
                    pos = cs[r] + (incl - mi)
                    plsc.store_scatter(
                        lst, [jnp.full((L,), r, jnp.int32), pos], pk,
                        mask=inb)
                    outs.append(cs[r] + plsc.all_reduce_population_count(inb))
                new = tuple(outs)
            return new

        with jax.named_scope("p1"):
            cv_a, cv_b = lax.fori_loop(
                0, P // L // 2, p1_body,
                (jnp.zeros((L,), jnp.int32), jnp.zeros((L,), jnp.int32)))
            cnt_a = jnp.max(cv_a)
            cnt_b = jnp.max(cv_b)

        def apply_roi(r, cnt):
            rs = jnp.full((L,), r, jnp.int32)
            ones = jnp.ones((L,), jnp.int32)

            def ch_body(c, carry):
                base = c * L
                pk = lst[r, pl.ds(base, L)]
                rem = cnt - base
                valid = lane < rem
                idxb[...] = jnp.where(valid, pk & (P - 1), 0)
                pltpu.async_copy(feat_hbm.at[idxb], rows, sem).wait()
                for j in range(L):
                    pkj = plsc.load_gather(
                        lst, [rs, jnp.full((L,), base + j, jnp.int32)])
                    voxj = jnp.minimum(
                        jnp.maximum(pkj >> pbits, 0), VOX - 1)
                    vj = jnp.full((L,), j, jnp.int32) < rem
                    idx0 = voxj * C + lane
                    idx1 = idx0 + L
                    hv = plsc.load_gather(hit, [rs, voxj])
                    g0 = plsc.load_gather(grid, [rs, idx0])
                    g1 = plsc.load_gather(grid, [rs, idx1])
                    f0 = rows[j, pl.ds(0, L)]
                    f1 = rows[j, pl.ds(L, L)]
                    hb = hv > 0
                    n0 = jnp.maximum(jnp.where(hb, g0, NEG), f0)
                    n1 = jnp.maximum(jnp.where(hb, g1, NEG), f1)
                    plsc.store_scatter(grid, [rs, idx0], n0, mask=vj)
                    plsc.store_scatter(grid, [rs, idx1], n1, mask=vj)
                    plsc.store_scatter(hit, [rs, voxj], ones, mask=vj)
                return carry

            nch = (cnt + (L - 1)) >> 4
            lax.fori_loop(0, nch, ch_body, 0)

        with jax.named_scope("p2"):
            apply_roi(0, cnt_a)
            apply_roi(1, cnt_b)

        with jax.named_scope("wout"):
            pltpu.sync_copy(grid.at[0], out_hbm.at[r0])
            pltpu.sync_copy(grid.at[1], out_hbm.at[r0 + 1])

    return body(params16, pts_t, pts_feature)


def kernel(rois, pts, pts_feature):
    n = rois.shape[0]
    p, c = pts_feature.shape
    cx, cy, cz = rois[:, 0], rois[:, 1], rois[:, 2]
    dx, dy, dz = rois[:, 3], rois[:, 4], rois[:, 5]
    rz = rois[:, 6]
    cosa = jnp.cos(-rz)
    sina = jnp.sin(-rz)
    maxr2 = (dx * dx + dy * dy) * jnp.float32(0.2500002) + jnp.float32(1e-5)
    params = jnp.stack(
        [cx, cy, cz, cosa, sina, dx * 0.5, dy * 0.5, dz,
         dx / OX, dy / OY, dz / OZ, maxr2], axis=1).astype(jnp.float32)
    pts_flat = pts.astype(jnp.float32).reshape(-1)
    pooled = _pool_sc(params, pts_flat, pts_feature.astype(jnp.float32), n)
    return pooled.reshape(n, OX, OY, OZ, c)


# speedup vs baseline: 1.1574x; 1.1574x over previous
import functools

import jax
import jax.numpy as jnp
from jax import lax
from jax.experimental import pallas as pl
from jax.experimental.pallas import tpu as pltpu
from jax.experimental.pallas import tpu_sc as plsc

OX, OY, OZ = 8, 8, 8
VOX = OX * OY * OZ
L = 16
NEG = -3.0e38


def _pool_sc(params16, pts_t, pts_feature, n_rois):
    P, C = pts_feature.shape
    assert C == 2 * L
    assert n_rois % 32 == 0
    rois_per_w = n_rois // 32
    assert rois_per_w == 2
    pbits = (P - 1).bit_length()
    mesh = plsc.VectorSubcoreMesh(core_axis_name="c", subcore_axis_name="s")

    @functools.partial(
        pl.kernel,
        out_type=jax.ShapeDtypeStruct((n_rois, VOX * C), jnp.float32),
        mesh=mesh,
        scratch_types=[
            pltpu.VMEM((2 * 16,), jnp.float32),
            pltpu.VMEM((3, P), jnp.float32),
            pltpu.VMEM((2, VOX * C), jnp.float32),
            pltpu.VMEM((2, VOX), jnp.int32),
            pltpu.VMEM((2, P), jnp.int32),
            pltpu.VMEM((L, C), jnp.float32),
            pltpu.VMEM((L,), jnp.int32),
            pltpu.SemaphoreType.DMA,
        ],
        compiler_params=pltpu.CompilerParams(
            needs_layout_passes=False, use_tc_tiling_on_sc=False),
    )
    def body(params_hbm, pts_hbm, feat_hbm, out_hbm,
             prm, ptsv, grid, hit, lst, rows, idxb, sem):
        wid = lax.axis_index("s") * 2 + lax.axis_index("c")
        r0 = wid * 2
        with jax.named_scope("stage"):
            pltpu.sync_copy(pts_hbm, ptsv)
            pltpu.sync_copy(params_hbm.at[pl.ds(r0 * 16, 2 * 16)], prm)

        lane = lax.iota(jnp.int32, L)
        lane3 = lane * 3
        zf = jnp.zeros((L,), jnp.float32)
        zi = jnp.zeros((L,), jnp.int32)

        def zero_grid(i, carry):
            base = i * (4 * L)
            for r in range(2):
                for u in range(4):
                    grid[r, pl.ds(base + u * L, L)] = zf
            return carry

        with jax.named_scope("zero"):
            lax.fori_loop(0, VOX * C // (4 * L), zero_grid, 0)

        def zero_hit(i, carry):
            hit[0, pl.ds(i * L, L)] = zi
            hit[1, pl.ds(i * L, L)] = zi
            return carry

        lax.fori_loop(0, VOX // L, zero_hit, 0)

        dn = lax.GatherDimensionNumbers(
            offset_dims=(), collapsed_slice_dims=(0,), start_index_map=(0,))

        def splat(vec, k):
            idx = jnp.full((L, 1), k, jnp.int32)
            return lax.gather(vec, idx, dn, (1,),
                              mode=lax.GatherScatterMode.PROMISE_IN_BOUNDS)

        pv = [prm[pl.ds(0, L)], prm[pl.ds(L, L)]]
        pr = [[splat(pv[r], k) for k in range(12)] for r in range(2)]

        def clampi(t, hi):
            ti = t.astype(jnp.int32)
            return jnp.minimum(jnp.maximum(ti, 0), hi)

        def p1_body(i, cnts):
            new = []
            for sub in range(2):
                ii = i * 2 + sub
                x = ptsv[0, pl.ds(ii * L, L)]
                y = ptsv[1, pl.ds(ii * L, L)]
                z = ptsv[2, pl.ds(ii * L, L)]
                pid = ii * L + lane

                pxs, pys, pzs, zoks = [], [], [], []
                for r in range(2):
                    cx, cy, cz, ca, sa, hdx, hdy, dz, xre, yre, zre, mr2 = pr[r]
                    px = x - cx
                    py = y - cy
                    pz = z - cz
                    zok = (pz > 0.0) & (pz < dz)
                    pxs.append(px); pys.append(py); pzs.append(pz)
                    zoks.append(zok)

                cs = cnts if sub == 0 else new
                outs = []
                for r in range(2):
                    cx, cy, cz, ca, sa, hdx, hdy, dz, xre, yre, zre, mr2 = pr[r]
                    px, py, pz = pxs[r], pys[r], pzs[r]
                    lx = px * ca - py * sa
                    ly = px * sa + py * ca
                    inb = ((jnp.abs(lx) < hdx) & (jnp.abs(ly) < hdy)
                           & zoks[r])
                    xi = clampi((lx + hdx) / xre, OX - 1)
                    yi = clampi((ly + hdy) / yre, OY - 1)
                    zvi = clampi(pz / zre, OZ - 1)
                    vox = (xi * OY + yi) * OZ + zvi
                    pk = (vox << pbits) | pid
                    mi = inb.astype(jnp.int32)
                    incl = plsc.cumsum(mi)
                    pos = cs[r] + (incl - mi)
                    plsc.store_scatter(
                        lst, [jnp.full((L,), r, jnp.int32), pos], pk,
                        mask=inb)
                    outs.append(cs[r] + plsc.all_reduce_population_count(inb))
                new = tuple(outs)
            return new

        with jax.named_scope("p1"):
            cv_a, cv_b = lax.fori_loop(
                0, P // L // 2, p1_body,
                (jnp.zeros((L,), jnp.int32), jnp.zeros((L,), jnp.int32)))
            cnt_a = jnp.max(cv_a)
            cnt_b = jnp.max(cv_b)

        def apply_roi(r, cnt):
            rs = jnp.full((L,), r, jnp.int32)
            ones = jnp.ones((L,), jnp.int32)

            def ch_body(c, carry):
                base = c * L
                pk = lst[r, pl.ds(base, L)]
                rem = cnt - base
                valid = lane < rem
                idxb[...] = jnp.where(valid, pk & (P - 1), 0)
                pltpu.async_copy(feat_hbm.at[idxb], rows, sem).wait()
                for j in range(L):
                    pkj = plsc.load_gather(
                        lst, [rs, jnp.full((L,), base + j, jnp.int32)])
                    voxj = jnp.minimum(
                        jnp.maximum(pkj >> pbits, 0), VOX - 1)
                    vj = jnp.full((L,), j, jnp.int32) < rem
                    idx0 = voxj * C + lane
                    idx1 = idx0 + L
                    hv = plsc.load_gather(hit, [rs, voxj])
                    g0 = plsc.load_gather(grid, [rs, idx0])
                    g1 = plsc.load_gather(grid, [rs, idx1])
                    f0 = rows[j, pl.ds(0, L)]
                    f1 = rows[j, pl.ds(L, L)]
                    hb = hv > 0
                    n0 = jnp.maximum(jnp.where(hb, g0, NEG), f0)
                    n1 = jnp.maximum(jnp.where(hb, g1, NEG), f1)
                    plsc.store_scatter(grid, [rs, idx0], n0, mask=vj)
                    plsc.store_scatter(grid, [rs, idx1], n1, mask=vj)
                    plsc.store_scatter(hit, [rs, voxj], ones, mask=vj)
                return carry

            nch = (cnt + (L - 1)) >> 4
            lax.fori_loop(0, nch, ch_body, 0)

        with jax.named_scope("p2"):
            apply_roi(0, cnt_a)
            apply_roi(1, cnt_b)

        with jax.named_scope("wout"):
            pltpu.sync_copy(grid.at[0], out_hbm.at[r0])
            pltpu.sync_copy(grid.at[1], out_hbm.at[r0 + 1])

    return body(params16, pts_t, pts_feature)


def kernel(rois, pts, pts_feature):
    n = rois.shape[0]
    p, c = pts_feature.shape
    cx, cy, cz = rois[:, 0], rois[:, 1], rois[:, 2]
    dx, dy, dz = rois[:, 3], rois[:, 4], rois[:, 5]
    rz = rois[:, 6]
    cosa = jnp.cos(-rz)
    sina = jnp.sin(-rz)
    maxr2 = (dx * dx + dy * dy) * jnp.float32(0.2500002) + jnp.float32(1e-5)
    zpad = jnp.zeros_like(cx)
    params = jnp.stack(
        [cx, cy, cz, cosa, sina, dx * 0.5, dy * 0.5, dz,
         dx / OX, dy / OY, dz / OZ, maxr2,
         zpad, zpad, zpad, zpad], axis=1).astype(jnp.float32).reshape(-1)
    pts_t = pts.T.astype(jnp.float32)
    pooled = _pool_sc(params, pts_t, pts_feature.astype(jnp.float32), n)
    return pooled.reshape(n, OX, OY, OZ, c)
